# baseline (speedup 1.0000x reference)
import functools

import jax
import jax.numpy as jnp
from jax.experimental import pallas as pl
from jax.experimental.pallas import tpu as pltpu

KS = 5
W1L = 192
W2L = 128
FCPAD = 128
NCLASS = 10


def _lenet_kernel(x_ref, t1_ref, bb1_ref, t2_ref, bb2_ref,
                  fw1_ref, fb1_ref, fw2_ref, fb2_ref, o_ref, *, tb):
    f32 = jnp.float32
    bf16 = jnp.bfloat16

    xt = jnp.transpose(x_ref[0], (1, 0, 2)).reshape(28 * tb, 28)

    xg = jnp.concatenate(
        [xt[kh * tb:(kh + 24) * tb] for kh in range(KS)], axis=1)

    c0 = jnp.dot(xg, t1_ref[0], preferred_element_type=f32)
    c1 = jnp.dot(xg, t1_ref[1], preferred_element_type=f32)
    full = jnp.maximum(c0, c1)
    bb1 = bb1_ref[...]
    p1 = []
    for rho in range(2):
        p1.append(jnp.concatenate(
            [jnp.maximum(
                jnp.maximum(full[(4 * g + 2 * rho) * tb:(4 * g + 2 * rho + 1) * tb],
                            full[(4 * g + 2 * rho + 1) * tb:(4 * g + 2 * rho + 2) * tb])
                + bb1, 0.0).astype(bf16)
             for g in range(6)], axis=0))

    mx = None
    for a2 in range(2):
        acc = None
        for kh in range(KS):
            t = a2 + kh
            lhs = p1[t % 2][(t // 2) * tb:(t // 2 + 4) * tb]
            part = jnp.dot(lhs, t2_ref[kh], preferred_element_type=f32)
            acc = part if acc is None else acc + part
        m_a = jnp.maximum(acc[:, :W2L], acc[:, W2L:])
        mx = m_a if mx is None else jnp.maximum(mx, m_a)
    pool2 = jnp.maximum(mx + bb2_ref[...], 0.0).astype(bf16)

    h = None
    for qh in range(4):
        part = jnp.dot(pool2[qh * tb:(qh + 1) * tb], fw1_ref[qh],
                       preferred_element_type=f32)
        h = part if h is None else h + part
    h = jnp.maximum(h + fb1_ref[...], 0.0).astype(bf16)
    z = jnp.dot(h, fw2_ref[...], preferred_element_type=f32) + fb2_ref[...]

    col = jax.lax.broadcasted_iota(jnp.int32, z.shape, 1)
    z = jnp.where(col < NCLASS, z, -1e30)
    s = z - jnp.max(z, axis=-1, keepdims=True)
    o_ref[...] = s - jnp.log(jnp.sum(jnp.exp(s), axis=-1, keepdims=True))


def kernel(x, T1, BB1, T2, BB2, FW1, FB1, FW2, FB2):
    B = x.shape[0]
    TB = min(128, ((B + 7) // 8) * 8)
    nt = pl.cdiv(B, TB)
    Bp = nt * TB

    xs = x.reshape(B, 28, 28).astype(jnp.bfloat16)
    if Bp != B:
        xs = jnp.pad(xs, ((0, Bp - B), (0, 0), (0, 0)))
    xh = xs.reshape(nt, TB, 28, 28)

    out = pl.pallas_call(
        functools.partial(_lenet_kernel, tb=TB),
        out_shape=jax.ShapeDtypeStruct((Bp, FCPAD), jnp.float32),
        grid=(nt,),
        in_specs=[
            pl.BlockSpec((1, TB, 28, 28), lambda i: (i, 0, 0, 0)),
            pl.BlockSpec(T1.shape, lambda i: (0, 0, 0)),
            pl.BlockSpec(BB1.shape, lambda i: (0, 0)),
            pl.BlockSpec(T2.shape, lambda i: (0, 0, 0)),
            pl.BlockSpec(BB2.shape, lambda i: (0, 0)),
            pl.BlockSpec(FW1.shape, lambda i: (0, 0, 0)),
            pl.BlockSpec(FB1.shape, lambda i: (0, 0)),
            pl.BlockSpec(FW2.shape, lambda i: (0, 0)),
            pl.BlockSpec(FB2.shape, lambda i: (0, 0)),
        ],
        out_specs=pl.BlockSpec((TB, FCPAD), lambda i: (i, 0)),
        compiler_params=pltpu.CompilerParams(
            dimension_semantics=("parallel",),
            vmem_limit_bytes=64 * 1024 * 1024),
    )(xh, T1, BB1, T2, BB2, FW1, FB1, FW2, FB2)
    return out[:B, :NCLASS]


# speedup vs baseline: 4.9622x; 4.9622x over previous
import functools

import jax
import jax.numpy as jnp
from jax.experimental import pallas as pl
from jax.experimental.pallas import tpu as pltpu

KS = 5
W1L = 192
W2L = 128
FCPAD = 128
NCLASS = 10


def _lenet_kernel(x_ref, t1_ref, bb1_ref, t2_ref, bb2_ref,
                  fw1_ref, fb1_ref, fw2_ref, fb2_ref, o_ref, *, tb):
    f32 = jnp.float32
    bf16 = jnp.bfloat16

    xt = jnp.transpose(x_ref[0].astype(jnp.bfloat16),
                       (1, 0, 2)).reshape(28 * tb, 28)

    xg = jnp.concatenate(
        [xt[kh * tb:(kh + 24) * tb] for kh in range(KS)], axis=1)

    c0 = jnp.dot(xg, t1_ref[0], preferred_element_type=f32)
    c1 = jnp.dot(xg, t1_ref[1], preferred_element_type=f32)
    full = jnp.maximum(c0, c1)
    bb1 = bb1_ref[...]
    p1 = []
    for rho in range(2):
        p1.append(jnp.concatenate(
            [jnp.maximum(
                jnp.maximum(full[(4 * g + 2 * rho) * tb:(4 * g + 2 * rho + 1) * tb],
                            full[(4 * g + 2 * rho + 1) * tb:(4 * g + 2 * rho + 2) * tb])
                + bb1, 0.0).astype(bf16)
             for g in range(6)], axis=0))

    mx = None
    for a2 in range(2):
        acc = None
        for kh in range(KS):
            t = a2 + kh
            lhs = p1[t % 2][(t // 2) * tb:(t // 2 + 4) * tb]
            part = jnp.dot(lhs, t2_ref[kh], preferred_element_type=f32)
            acc = part if acc is None else acc + part
        m_a = jnp.maximum(acc[:, :W2L], acc[:, W2L:])
        mx = m_a if mx is None else jnp.maximum(mx, m_a)
    pool2 = jnp.maximum(mx + bb2_ref[...], 0.0).astype(bf16)

    h = None
    for qh in range(4):
        part = jnp.dot(pool2[qh * tb:(qh + 1) * tb], fw1_ref[qh],
                       preferred_element_type=f32)
        h = part if h is None else h + part
    h = jnp.maximum(h + fb1_ref[...], 0.0).astype(bf16)
    z = jnp.dot(h, fw2_ref[...], preferred_element_type=f32) + fb2_ref[...]

    col = jax.lax.broadcasted_iota(jnp.int32, z.shape, 1)
    z = jnp.where(col < NCLASS, z, -1e30)
    s = z - jnp.max(z, axis=-1, keepdims=True)
    o_ref[...] = s - jnp.log(jnp.sum(jnp.exp(s), axis=-1, keepdims=True))


def kernel(x, T1, BB1, T2, BB2, FW1, FB1, FW2, FB2):
    B = x.shape[0]
    TB = min(128, ((B + 7) // 8) * 8)
    nt = pl.cdiv(B, TB)
    Bp = nt * TB

    xs = x.reshape(B, 28, 28)
    if Bp != B:
        xs = jnp.pad(xs, ((0, Bp - B), (0, 0), (0, 0)))
    xh = xs.reshape(nt, TB, 28, 28)

    out = pl.pallas_call(
        functools.partial(_lenet_kernel, tb=TB),
        out_shape=jax.ShapeDtypeStruct((Bp, FCPAD), jnp.float32),
        grid=(nt,),
        in_specs=[
            pl.BlockSpec((1, TB, 28, 28), lambda i: (i, 0, 0, 0)),
            pl.BlockSpec(T1.shape, lambda i: (0, 0, 0)),
            pl.BlockSpec(BB1.shape, lambda i: (0, 0)),
            pl.BlockSpec(T2.shape, lambda i: (0, 0, 0)),
            pl.BlockSpec(BB2.shape, lambda i: (0, 0)),
            pl.BlockSpec(FW1.shape, lambda i: (0, 0, 0)),
            pl.BlockSpec(FB1.shape, lambda i: (0, 0)),
            pl.BlockSpec(FW2.shape, lambda i: (0, 0)),
            pl.BlockSpec(FB2.shape, lambda i: (0, 0)),
        ],
        out_specs=pl.BlockSpec((TB, FCPAD), lambda i: (i, 0)),
        compiler_params=pltpu.CompilerParams(
            dimension_semantics=("parallel",),
            vmem_limit_bytes=64 * 1024 * 1024),
    )(xh, T1, BB1, T2, BB2, FW1, FB1, FW2, FB2)
    return out[:B, :NCLASS]


# speedup vs baseline: 5.4523x; 1.0988x over previous
import functools

import jax
import jax.numpy as jnp
from jax.experimental import pallas as pl
from jax.experimental.pallas import tpu as pltpu

KS = 5
W1L = 192
W2L = 128
FCPAD = 128
NCLASS = 10


def _lenet_kernel(x_ref, t1_ref, bb1_ref, t2_ref, bb2_ref,
                  fw1_ref, fb1_ref, fw2_ref, fb2_ref, o_ref, *, tb):
    f32 = jnp.float32
    bf16 = jnp.bfloat16

    xt = jnp.transpose(x_ref[0].astype(jnp.bfloat16),
                       (1, 0, 2)).reshape(28 * tb, 28)

    xg = jnp.concatenate(
        [xt[kh * tb:(kh + 24) * tb] for kh in range(KS)], axis=1)

    c0 = jnp.dot(xg, t1_ref[0], preferred_element_type=f32)
    c1 = jnp.dot(xg, t1_ref[1], preferred_element_type=f32)
    full = jnp.maximum(c0, c1)
    bb1 = bb1_ref[...]
    p1 = []
    for rho in range(2):
        p1.append(jnp.concatenate(
            [jnp.maximum(
                jnp.maximum(full[(4 * g + 2 * rho) * tb:(4 * g + 2 * rho + 1) * tb],
                            full[(4 * g + 2 * rho + 1) * tb:(4 * g + 2 * rho + 2) * tb])
                + bb1, 0.0).astype(bf16)
             for g in range(6)], axis=0))

    mx = None
    for a2 in range(2):
        acc = None
        for kh in range(KS):
            t = a2 + kh
            lhs = p1[t % 2][(t // 2) * tb:(t // 2 + 4) * tb]
            part = jnp.dot(lhs, t2_ref[kh], preferred_element_type=f32)
            acc = part if acc is None else acc + part
        m_a = jnp.maximum(acc[:, :W2L], acc[:, W2L:])
        mx = m_a if mx is None else jnp.maximum(mx, m_a)
    pool2 = jnp.maximum(mx + bb2_ref[...], 0.0).astype(bf16)

    fcx = jnp.concatenate(
        [pool2[qh * tb:(qh + 1) * tb] for qh in range(4)], axis=1)
    h = jnp.dot(fcx, fw1_ref[...].reshape(4 * W2L, FCPAD),
                preferred_element_type=f32)
    h = jnp.maximum(h + fb1_ref[...], 0.0).astype(bf16)
    z = jnp.dot(h, fw2_ref[...], preferred_element_type=f32) + fb2_ref[...]

    col = jax.lax.broadcasted_iota(jnp.int32, z.shape, 1)
    z = jnp.where(col < NCLASS, z, -1e30)
    s = z - jnp.max(z, axis=-1, keepdims=True)
    o_ref[...] = s - jnp.log(jnp.sum(jnp.exp(s), axis=-1, keepdims=True))


def kernel(x, T1, BB1, T2, BB2, FW1, FB1, FW2, FB2):
    B = x.shape[0]
    TB = min(256, ((B + 7) // 8) * 8)
    nt = pl.cdiv(B, TB)
    Bp = nt * TB

    xs = x.reshape(B, 28, 28)
    if Bp != B:
        xs = jnp.pad(xs, ((0, Bp - B), (0, 0), (0, 0)))
    xh = xs.reshape(nt, TB, 28, 28)

    out = pl.pallas_call(
        functools.partial(_lenet_kernel, tb=TB),
        out_shape=jax.ShapeDtypeStruct((Bp, FCPAD), jnp.float32),
        grid=(nt,),
        in_specs=[
            pl.BlockSpec((1, TB, 28, 28), lambda i: (i, 0, 0, 0)),
            pl.BlockSpec(T1.shape, lambda i: (0, 0, 0)),
            pl.BlockSpec(BB1.shape, lambda i: (0, 0)),
            pl.BlockSpec(T2.shape, lambda i: (0, 0, 0)),
            pl.BlockSpec(BB2.shape, lambda i: (0, 0)),
            pl.BlockSpec(FW1.shape, lambda i: (0, 0, 0)),
            pl.BlockSpec(FB1.shape, lambda i: (0, 0)),
            pl.BlockSpec(FW2.shape, lambda i: (0, 0)),
            pl.BlockSpec(FB2.shape, lambda i: (0, 0)),
        ],
        out_specs=pl.BlockSpec((TB, FCPAD), lambda i: (i, 0)),
        compiler_params=pltpu.CompilerParams(
            dimension_semantics=("parallel",),
            vmem_limit_bytes=64 * 1024 * 1024),
    )(xh, T1, BB1, T2, BB2, FW1, FB1, FW2, FB2)
    return out[:B, :NCLASS]


# speedup vs baseline: 5.6573x; 1.0376x over previous
import functools

import jax
import jax.numpy as jnp
from jax.experimental import pallas as pl
from jax.experimental.pallas import tpu as pltpu

KS = 5
W1L = 192
W2L = 128
FCPAD = 128
NCLASS = 10


def _lenet_kernel(x_ref, t1_ref, bb1_ref, t2_ref, bb2_ref,
                  fw1_ref, fb1_ref, fw2_ref, fb2_ref, o_ref, *, tb):
    f32 = jnp.float32
    bf16 = jnp.bfloat16

    xt = jnp.transpose(x_ref[0].astype(jnp.bfloat16),
                       (1, 0, 2)).reshape(28 * tb, 28)

    xg = jnp.concatenate(
        [xt[kh * tb:(kh + 24) * tb] for kh in range(KS)], axis=1)

    c0 = jnp.dot(xg, t1_ref[0], preferred_element_type=f32)
    c1 = jnp.dot(xg, t1_ref[1], preferred_element_type=f32)
    full = jnp.maximum(c0, c1)
    bb1 = bb1_ref[...]
    p1 = []
    for rho in range(2):
        p1.append(jnp.concatenate(
            [jnp.maximum(
                jnp.maximum(full[(4 * g + 2 * rho) * tb:(4 * g + 2 * rho + 1) * tb],
                            full[(4 * g + 2 * rho + 1) * tb:(4 * g + 2 * rho + 2) * tb])
                + bb1, 0.0).astype(bf16)
             for g in range(6)], axis=0))

    mx = None
    for a2 in range(2):
        acc = None
        for kh in range(KS):
            t = a2 + kh
            lhs = p1[t % 2][(t // 2) * tb:(t // 2 + 4) * tb]
            part = jnp.dot(lhs, t2_ref[kh], preferred_element_type=f32)
            acc = part if acc is None else acc + part
        m_a = jnp.maximum(acc[:, :W2L], acc[:, W2L:])
        mx = m_a if mx is None else jnp.maximum(mx, m_a)
    pool2 = jnp.maximum(mx + bb2_ref[...], 0.0).astype(bf16)

    fcx = jnp.concatenate(
        [pool2[qh * tb:(qh + 1) * tb] for qh in range(4)], axis=1)
    h = jnp.dot(fcx, fw1_ref[...].reshape(4 * W2L, FCPAD),
                preferred_element_type=f32)
    h = jnp.maximum(h + fb1_ref[...], 0.0).astype(bf16)
    z = jnp.dot(h, fw2_ref[...], preferred_element_type=f32) + fb2_ref[...]

    col = jax.lax.broadcasted_iota(jnp.int32, z.shape, 1)
    z = jnp.where(col < NCLASS, z, -1e30)
    s = z - jnp.max(z, axis=-1, keepdims=True)
    o_ref[...] = s - jnp.log(jnp.sum(jnp.exp(s), axis=-1, keepdims=True))


def kernel(x, T1, BB1, T2, BB2, FW1, FB1, FW2, FB2):
    B = x.shape[0]
    TB = min(512, ((B + 7) // 8) * 8)
    nt = pl.cdiv(B, TB)
    Bp = nt * TB

    xs = x.reshape(B, 28, 28)
    if Bp != B:
        xs = jnp.pad(xs, ((0, Bp - B), (0, 0), (0, 0)))
    xh = xs.reshape(nt, TB, 28, 28)

    out = pl.pallas_call(
        functools.partial(_lenet_kernel, tb=TB),
        out_shape=jax.ShapeDtypeStruct((Bp, FCPAD), jnp.float32),
        grid=(nt,),
        in_specs=[
            pl.BlockSpec((1, TB, 28, 28), lambda i: (i, 0, 0, 0)),
            pl.BlockSpec(T1.shape, lambda i: (0, 0, 0)),
            pl.BlockSpec(BB1.shape, lambda i: (0, 0)),
            pl.BlockSpec(T2.shape, lambda i: (0, 0, 0)),
            pl.BlockSpec(BB2.shape, lambda i: (0, 0)),
            pl.BlockSpec(FW1.shape, lambda i: (0, 0, 0)),
            pl.BlockSpec(FB1.shape, lambda i: (0, 0)),
            pl.BlockSpec(FW2.shape, lambda i: (0, 0)),
            pl.BlockSpec(FB2.shape, lambda i: (0, 0)),
        ],
        out_specs=pl.BlockSpec((TB, FCPAD), lambda i: (i, 0)),
        compiler_params=pltpu.CompilerParams(
            dimension_semantics=("parallel",),
            vmem_limit_bytes=64 * 1024 * 1024),
    )(xh, T1, BB1, T2, BB2, FW1, FB1, FW2, FB2)
    return out[:B, :NCLASS]
